# baseline (speedup 1.0000x reference)
import functools

import jax
import jax.numpy as jnp
from jax import lax
from jax.experimental import pallas as pl
from jax.experimental.pallas import tpu as pltpu
from jax.experimental.pallas import tpu_sc as plsc

D = 128
NN = 4096
NP = 3
LANES = 16
NC = 2
NS = 16
NW = NC * NS
RPW = NN // NW
NG = RPW // LANES
MC = D // LANES


def _sc_partials(task, idx0, idx1, idx2, emb, w_stack):
  mesh = plsc.VectorSubcoreMesh(core_axis_name="c", subcore_axis_name="s")

  @functools.partial(
      pl.kernel,
      out_type=[
          jax.ShapeDtypeStruct((NW, NP, D), jnp.float32),
          jax.ShapeDtypeStruct((NW, NP, LANES), jnp.float32),
      ],
      mesh=mesh,
      compiler_params=pltpu.CompilerParams(needs_layout_passes=False),
      scratch_types=[
          pltpu.VMEM((RPW,), jnp.int32),
          pltpu.VMEM((RPW + LANES, D), jnp.float32),
          pltpu.VMEM((NP, 2 * D), jnp.float32),
          pltpu.VMEM((NP, D), jnp.float32),
          pltpu.VMEM((NP, LANES), jnp.float32),
          pltpu.SemaphoreType.DMA,
      ],
  )
  def sc_kernel(task_hbm, i0, i1, i2, emb_hbm, w_hbm, wsum_out, z_out,
                idx_v, rows_v, w_v, acc_v, z_v, sem):
    cid = lax.axis_index("c")
    sid = lax.axis_index("s")
    wid = sid * NC + cid
    base = wid * RPW
    pltpu.sync_copy(w_hbm, w_v)
    for j in range(LANES):
      pltpu.sync_copy(task_hbm, rows_v.at[RPW + j])

    def dot16(ridx, p, half):
      s = jnp.zeros((LANES,), jnp.float32)
      for m in range(MC):
        wchunk = w_v[p, pl.ds(half * D + m * LANES, LANES)]
        for t in range(LANES):
          k = m * LANES + t
          col = plsc.load_gather(
              rows_v, [ridx, jnp.full((LANES,), k, jnp.int32)])
          s = s + col * wchunk[t]
      return s

    tidx = RPW + lax.iota(jnp.int32, LANES)
    for p, idx_hbm in enumerate((i0, i1, i2)):
      pltpu.sync_copy(idx_hbm.at[pl.ds(base, RPW)], idx_v)
      gat = pltpu.async_copy(emb_hbm.at[idx_v], rows_v.at[pl.ds(0, RPW)], sem)
      cvec = dot16(tidx, p, 0)
      gat.wait()

      def gbody(g, carry):
        accs, zacc = carry
        ridx = g * LANES + lax.iota(jnp.int32, LANES)
        s = dot16(ridx, p, 1)
        sig = 1.0 / (1.0 + jnp.exp(-(cvec + s)))
        e = jnp.exp(sig)
        zacc = zacc + e
        new_accs = list(accs)
        for j in range(LANES):
          row = g * LANES + j
          ej = e[j]
          for m in range(MC):
            new_accs[m] = new_accs[m] + ej * rows_v[row, pl.ds(m * LANES, LANES)]
        return tuple(new_accs), zacc

      init = (tuple(jnp.zeros((LANES,), jnp.float32) for _ in range(MC)),
              jnp.zeros((LANES,), jnp.float32))
      accs, zacc = lax.fori_loop(0, NG, gbody, init)
      for m in range(MC):
        acc_v[p, pl.ds(m * LANES, LANES)] = accs[m]
      z_v[p, ...] = zacc

    pltpu.sync_copy(acc_v, wsum_out.at[wid])
    pltpu.sync_copy(z_v, z_out.at[wid])

  return sc_kernel(task, idx0, idx1, idx2, emb, w_stack)


def _tc_finish(wsum, zpart, task2d, w1, w2, b2d, v):
  def body(wsum_ref, z_ref, task_ref, w1_ref, w2_ref, b_ref, v_ref, out_ref):
    paths_raw = jnp.sum(wsum_ref[...], axis=0)
    z = jnp.sum(z_ref[...], axis=0)
    zden = jnp.sum(z, axis=1, keepdims=True)
    paths = paths_raw / zden
    q = jnp.dot(task_ref[...], w1_ref[...])
    t = jnp.tanh(q + jnp.dot(paths, w2_ref[...]) + b_ref[...])
    logits = jnp.dot(t, v_ref[...])
    e = jnp.exp(logits - jnp.max(logits))
    sw = e / jnp.sum(e)
    out_ref[...] = jnp.sum(paths * sw, axis=0, keepdims=True)

  return pl.pallas_call(
      body,
      out_shape=jax.ShapeDtypeStruct((1, D), jnp.float32),
  )(wsum, zpart, task2d, w1, w2, b2d, v)


def kernel(task_repre, neighbors_p0, neighbors_p1, neighbors_p2, emb_table,
           W_p0, W_p1, W_p2, w1, w2, b, v):
  i0 = neighbors_p0.astype(jnp.int32)
  i1 = neighbors_p1.astype(jnp.int32)
  i2 = neighbors_p2.astype(jnp.int32)
  w_stack = jnp.concatenate([W_p0, W_p1, W_p2], axis=0)
  wsum, zpart = _sc_partials(task_repre, i0, i1, i2, emb_table, w_stack)
  out = _tc_finish(wsum, zpart, task_repre.reshape(1, D), w1, w2,
                   b.reshape(1, D), v)
  return out.reshape(D)


# speedup vs baseline: 1.9003x; 1.9003x over previous
import functools

import jax
import jax.numpy as jnp
from jax import lax
from jax.experimental import pallas as pl
from jax.experimental.pallas import tpu as pltpu
from jax.experimental.pallas import tpu_sc as plsc

D = 128
NN = 4096
NP = 3
LANES = 16
NC = 2
NS = 16
NW = NC * NS
RPW = NN // NW
NG = RPW // LANES
MC = D // LANES
TS = LANES + 1
PW = D + LANES


def _sc_partials(task_rows, idx_all, emb, w_stack):
  mesh = plsc.VectorSubcoreMesh(core_axis_name="c", subcore_axis_name="s")

  @functools.partial(
      pl.kernel,
      out_type=jax.ShapeDtypeStruct((NW, NP, PW), jnp.float32),
      mesh=mesh,
      compiler_params=pltpu.CompilerParams(needs_layout_passes=False),
      scratch_types=[
          pltpu.VMEM((NP, RPW), jnp.int32),
          pltpu.VMEM((RPW, D), jnp.float32),
          pltpu.VMEM((RPW, D), jnp.float32),
          pltpu.VMEM((LANES, D), jnp.float32),
          pltpu.VMEM((NP, 2 * D), jnp.float32),
          pltpu.VMEM((LANES * TS,), jnp.float32),
          pltpu.VMEM((NP, PW), jnp.float32),
          pltpu.SemaphoreType.DMA,
          pltpu.SemaphoreType.DMA,
      ],
  )
  def sc_kernel(task_hbm, idx_hbm, emb_hbm, w_hbm, part_out,
                idx_v, rows_a, rows_b, task_v, w_v, tbuf, acc_v, sem_a, sem_b):
    cid = lax.axis_index("c")
    sid = lax.axis_index("s")
    wid = sid * NC + cid
    base = wid * RPW
    pltpu.sync_copy(w_hbm, w_v)
    pltpu.sync_copy(task_hbm, task_v)
    pltpu.sync_copy(idx_hbm.at[:, pl.ds(base, RPW)], idx_v)

    bufs = (rows_a, rows_b)
    sems = (sem_a, sem_b)
    jv = lax.iota(jnp.int32, LANES)
    jv17 = jv * TS

    def dot16(src, row_base, wch):
      for j in range(LANES):
        t = src[row_base + j, pl.ds(0, LANES)] * wch[0]
        for m in range(1, MC):
          t = t + src[row_base + j, pl.ds(m * LANES, LANES)] * wch[m]
        plsc.store_scatter(tbuf, [jv + (j * TS)], t)
      s = plsc.load_gather(tbuf, [jv17])
      for k in range(1, LANES):
        s = s + plsc.load_gather(tbuf, [jv17 + k])
      return s

    gat = pltpu.async_copy(emb_hbm.at[idx_v.at[0]], rows_a, sem_a)

    cvecs = []
    for p in range(NP):
      wch0 = [w_v[p, pl.ds(m * LANES, LANES)] for m in range(MC)]
      cvecs.append(dot16(task_v, 0, wch0))

    for p in range(NP):
      if p + 1 < NP:
        nxt = pltpu.async_copy(
            emb_hbm.at[idx_v.at[p + 1]], bufs[(p + 1) % 2], sems[(p + 1) % 2])
      gat.wait()
      rows_v = bufs[p % 2]
      cvec = cvecs[p]
      wch1 = [w_v[p, pl.ds(D + m * LANES, LANES)] for m in range(MC)]

      def gbody(g, carry, rows_v=rows_v, cvec=cvec, wch1=wch1):
        accs, zacc = carry
        gbase = g * LANES
        s = dot16(rows_v, gbase, wch1)
        sig = 1.0 / (1.0 + jnp.exp(-(cvec + s)))
        e = jnp.exp(sig)
        zacc = zacc + e
        new_accs = list(accs)
        for j in range(LANES):
          ej = e[j]
          for m in range(MC):
            new_accs[m] = (
                new_accs[m] + ej * rows_v[gbase + j, pl.ds(m * LANES, LANES)])
        return tuple(new_accs), zacc

      init = (tuple(jnp.zeros((LANES,), jnp.float32) for _ in range(MC)),
              jnp.zeros((LANES,), jnp.float32))
      accs, zacc = lax.fori_loop(0, NG, gbody, init)
      for m in range(MC):
        acc_v[p, pl.ds(m * LANES, LANES)] = accs[m]
      acc_v[p, pl.ds(D, LANES)] = zacc
      if p + 1 < NP:
        gat = nxt

    pltpu.sync_copy(acc_v, part_out.at[wid])

  return sc_kernel(task_rows, idx_all, emb, w_stack)


def _tc_finish(part, task2d, w1, w2, b2d, v):
  def body(part_ref, task_ref, w1_ref, w2_ref, b_ref, v_ref, out_ref):
    part = part_ref[...]
    red = jnp.sum(part, axis=0)
    paths_raw = red[:, :D]
    zden = jnp.sum(red[:, D:], axis=1, keepdims=True)
    paths = paths_raw / zden
    q = jnp.dot(task_ref[...], w1_ref[...])
    t = jnp.tanh(q + jnp.dot(paths, w2_ref[...]) + b_ref[...])
    logits = jnp.dot(t, v_ref[...])
    e = jnp.exp(logits - jnp.max(logits))
    sw = e / jnp.sum(e)
    out_ref[...] = jnp.sum(paths * sw, axis=0, keepdims=True)

  return pl.pallas_call(
      body,
      out_shape=jax.ShapeDtypeStruct((1, D), jnp.float32),
  )(part, task2d, w1, w2, b2d, v)


def kernel(task_repre, neighbors_p0, neighbors_p1, neighbors_p2, emb_table,
           W_p0, W_p1, W_p2, w1, w2, b, v):
  idx_all = jnp.stack([neighbors_p0, neighbors_p1, neighbors_p2]
                      ).astype(jnp.int32)
  w_stack = jnp.concatenate([W_p0, W_p1, W_p2], axis=0)
  task_rows = jnp.tile(task_repre[None, :], (LANES, 1))
  part = _sc_partials(task_rows, idx_all, emb_table, w_stack)
  out = _tc_finish(part, task_repre.reshape(1, D), w1, w2,
                   b.reshape(1, D), v)
  return out.reshape(D)
